# baseline (speedup 1.0000x reference)
import functools

import jax
import jax.numpy as jnp
from jax import lax
from jax.experimental import pallas as pl
from jax.experimental.pallas import tpu as pltpu
from jax.experimental.pallas import tpu_sc as plsc

_B, _T, _C = 128, 196, 768
_M = 4
_S = _T // _M
_H = _B // 2
_COV = _H + 8
_NVC = _C // 16


def _sc_permute_t(xt):
    info = plsc.get_sparse_core_info()
    nw = info.num_cores * info.num_subcores
    wpq = nw // _M
    nk = (_S + wpq - 1) // wpq
    mesh = plsc.VectorSubcoreMesh(core_axis_name="c", subcore_axis_name="s")

    @functools.partial(
        pl.kernel,
        out_type=jax.ShapeDtypeStruct((_T, _B, _C), jnp.float32),
        mesh=mesh,
        scratch_types=[
            pltpu.VMEM((2, _COV, _C), jnp.float32),
            pltpu.SemaphoreType.DMA,
            pltpu.SemaphoreType.DMA,
        ],
    )
    def k(x_hbm, out_hbm, bufs, sem_in, sem_out):
        wid = lax.axis_index("s") * info.num_cores + lax.axis_index("c")
        mm = wid // wpq
        v = lax.rem(wid, wpq)
        units = [(kk, h) for kk in range(nk) for h in range(2)]

        def plane(kk):
            return _S * mm + lax.min(v + wpq * kk, _S - 1)

        def in_copies(j):
            kk, h = units[j]
            t = plane(kk)
            return [
                pltpu.make_async_copy(
                    x_hbm.at[t, pl.ds(_H * h, _H)],
                    bufs.at[j % 2, pl.ds(0, _H)], sem_in),
                pltpu.make_async_copy(
                    x_hbm.at[t, pl.ds((_H * h + _H) % _B, 8)],
                    bufs.at[j % 2, pl.ds(_H, 8)], sem_in),
            ]

        def shift(j):
            def body(r, carry):
                for vv in range(_NVC):
                    bufs[j % 2, r, pl.ds(vv * 16, 16)] = (
                        bufs[j % 2, r + mm, pl.ds(vv * 16, 16)])
                return carry

            lax.fori_loop(0, _H, body, 0)

        def out_copy(j):
            kk, h = units[j]
            return pltpu.make_async_copy(
                bufs.at[j % 2, pl.ds(0, _H)],
                out_hbm.at[plane(kk), pl.ds(_H * h, _H)], sem_out)

        n = len(units)
        ins = [in_copies(j) for j in range(n)]
        outs = [out_copy(j) for j in range(n)]
        for c in ins[0]:
            c.start()
        for j in range(n):
            for c in ins[j]:
                c.wait()
            shift(j)
            if j >= 1:
                outs[j - 1].wait()
            if j + 1 < n:
                for c in ins[j + 1]:
                    c.start()
            outs[j].start()
        outs[n - 1].wait()

    return k(xt)


def kernel(patches, m):
    del m
    xt = jnp.transpose(patches, (1, 0, 2))
    mixed = jnp.transpose(_sc_permute_t(xt), (1, 0, 2))
    ids_b = jnp.arange(_B).reshape(-1, 1)
    target = (ids_b + jnp.arange(_M)) % _B
    mix_target = (ids_b - _M + 1 + jnp.arange(_M * 2 - 1) + _B) % _B
    return (mixed, target, mix_target)


# speedup vs baseline: 1.1972x; 1.1972x over previous
import functools

import jax
import jax.numpy as jnp
from jax import lax
from jax.experimental import pallas as pl
from jax.experimental.pallas import tpu as pltpu
from jax.experimental.pallas import tpu_sc as plsc

_B, _T, _C = 128, 196, 768
_M = 4
_S = _T // _M
_H = _B // 2
_COV = _H + 8
_NVC = _C // 16


def _sc_permute_t(xt):
    info = plsc.get_sparse_core_info()
    nw = info.num_cores * info.num_subcores
    wpq = nw // _M
    nk = (_S + wpq - 1) // wpq
    mesh = plsc.VectorSubcoreMesh(core_axis_name="c", subcore_axis_name="s")

    @functools.partial(
        pl.kernel,
        out_type=jax.ShapeDtypeStruct((_T, _B, _C), jnp.float32),
        mesh=mesh,
        scratch_types=[
            pltpu.VMEM((2, _COV, _C), jnp.float32),
            pltpu.SemaphoreType.DMA,
            pltpu.SemaphoreType.DMA,
        ],
    )
    def k(x_hbm, out_hbm, bufs, sem_in, sem_out):
        wid = lax.axis_index("s") * info.num_cores + lax.axis_index("c")
        mm = wid // wpq
        v = lax.rem(wid, wpq)
        units = [(kk, h) for kk in range(nk) for h in range(2)]

        def plane(kk):
            return _S * mm + lax.min(v + wpq * kk, _S - 1)

        def in_copies(j):
            kk, h = units[j]
            t = plane(kk)
            return [
                pltpu.make_async_copy(
                    x_hbm.at[t, pl.ds(_H * h, _H)],
                    bufs.at[j % 2, pl.ds(0, _H)], sem_in),
                pltpu.make_async_copy(
                    x_hbm.at[t, pl.ds((_H * h + _H) % _B, 8)],
                    bufs.at[j % 2, pl.ds(_H, 8)], sem_in),
            ]

        def shift(j):
            def body(r, carry):
                src = bufs.at[j % 2, r + mm]
                dst = bufs.at[j % 2, r]
                for vv in range(_NVC):
                    dst[pl.ds(vv * 16, 16)] = src[pl.ds(vv * 16, 16)]
                return carry

            lax.fori_loop(0, _H, body, 0)

        def out_copy(j):
            kk, h = units[j]
            return pltpu.make_async_copy(
                bufs.at[j % 2, pl.ds(0, _H)],
                out_hbm.at[plane(kk), pl.ds(_H * h, _H)], sem_out)

        n = len(units)
        ins = [in_copies(j) for j in range(n)]
        outs = [out_copy(j) for j in range(n)]
        for c in ins[0]:
            c.start()
        for j in range(n):
            for c in ins[j]:
                c.wait()
            if j >= 1:
                outs[j - 1].wait()
            if j + 1 < n:
                for c in ins[j + 1]:
                    c.start()
            shift(j)
            outs[j].start()
        outs[n - 1].wait()

    return k(xt)


def kernel(patches, m):
    del m
    xt = jnp.transpose(patches, (1, 0, 2))
    mixed = jnp.transpose(_sc_permute_t(xt), (1, 0, 2))
    ids_b = jnp.arange(_B).reshape(-1, 1)
    target = (ids_b + jnp.arange(_M)) % _B
    mix_target = (ids_b - _M + 1 + jnp.arange(_M * 2 - 1) + _B) % _B
    return (mixed, target, mix_target)


# speedup vs baseline: 1.2236x; 1.0220x over previous
import functools

import jax
import jax.numpy as jnp
from jax import lax
from jax.experimental import pallas as pl
from jax.experimental.pallas import tpu as pltpu
from jax.experimental.pallas import tpu_sc as plsc

_B, _T, _C = 128, 196, 768
_M = 4
_S = _T // _M
_H = _B // 2
_COV = _H + 8
_NVC = _C // 16


def _sc_permute_t(xt):
    info = plsc.get_sparse_core_info()
    nw = info.num_cores * info.num_subcores
    wpq = nw // _M
    nk = (_S + wpq - 1) // wpq
    mesh = plsc.VectorSubcoreMesh(core_axis_name="c", subcore_axis_name="s")

    @functools.partial(
        pl.kernel,
        out_type=jax.ShapeDtypeStruct((_T, _B, _C), jnp.float32),
        mesh=mesh,
        scratch_types=[
            pltpu.VMEM((2, _COV, _C), jnp.float32),
            pltpu.SemaphoreType.DMA,
            pltpu.SemaphoreType.DMA,
        ],
    )
    def k(x_hbm, out_hbm, bufs, sem_in, sem_out):
        wid = lax.axis_index("s") * info.num_cores + lax.axis_index("c")
        mm = wid // wpq
        v = lax.rem(wid, wpq)
        units = [(kk, h) for kk in range(nk) for h in range(2)]

        def plane(kk):
            return _S * mm + lax.min(v + wpq * kk, _S - 1)

        def in_copies(j):
            kk, h = units[j]
            t = plane(kk)
            return [
                pltpu.make_async_copy(
                    x_hbm.at[t, pl.ds(_H * h, _H)],
                    bufs.at[j % 2, pl.ds(0, _H)], sem_in),
                pltpu.make_async_copy(
                    x_hbm.at[t, pl.ds((_H * h + _H) % _B, 8)],
                    bufs.at[j % 2, pl.ds(_H, 8)], sem_in),
            ]

        def shift(j):
            def body(r, carry):
                src = bufs.at[j % 2, r + mm]
                dst = bufs.at[j % 2, r]
                for vv in range(_NVC):
                    dst[pl.ds(vv * 16, 16)] = src[pl.ds(vv * 16, 16)]
                return carry

            lax.fori_loop(0, _H, body, 0, unroll=4)

        def out_copy(j):
            kk, h = units[j]
            return pltpu.make_async_copy(
                bufs.at[j % 2, pl.ds(0, _H)],
                out_hbm.at[plane(kk), pl.ds(_H * h, _H)], sem_out)

        n = len(units)
        ins = [in_copies(j) for j in range(n)]
        outs = [out_copy(j) for j in range(n)]
        for c in ins[0]:
            c.start()
        for j in range(n):
            for c in ins[j]:
                c.wait()
            if j >= 1:
                outs[j - 1].wait()
            if j + 1 < n:
                for c in ins[j + 1]:
                    c.start()
            shift(j)
            outs[j].start()
        outs[n - 1].wait()

    return k(xt)


def kernel(patches, m):
    del m
    xt = jnp.transpose(patches, (1, 0, 2))
    mixed = jnp.transpose(_sc_permute_t(xt), (1, 0, 2))
    ids_b = jnp.arange(_B).reshape(-1, 1)
    target = (ids_b + jnp.arange(_M)) % _B
    mix_target = (ids_b - _M + 1 + jnp.arange(_M * 2 - 1) + _B) % _B
    return (mixed, target, mix_target)


# speedup vs baseline: 1.2500x; 1.0216x over previous
import functools

import jax
import jax.numpy as jnp
from jax import lax
from jax.experimental import pallas as pl
from jax.experimental.pallas import tpu as pltpu
from jax.experimental.pallas import tpu_sc as plsc

_B, _T, _C = 128, 196, 768
_M = 4
_S = _T // _M
_H = _B // 4
_NQ = _B // _H
_COV = _H + 8
_NVC = _C // 16


def _sc_permute_t(xt):
    info = plsc.get_sparse_core_info()
    nw = info.num_cores * info.num_subcores
    wpq = nw // _M
    nk = (_S + wpq - 1) // wpq
    mesh = plsc.VectorSubcoreMesh(core_axis_name="c", subcore_axis_name="s")

    @functools.partial(
        pl.kernel,
        out_type=jax.ShapeDtypeStruct((_T, _B, _C), jnp.float32),
        mesh=mesh,
        scratch_types=[
            pltpu.VMEM((2, _COV, _C), jnp.float32),
            pltpu.VMEM((2, _H, _C), jnp.float32),
            pltpu.SemaphoreType.DMA,
            pltpu.SemaphoreType.DMA,
        ],
    )
    def k(x_hbm, out_hbm, bufs, stage, sem_in, sem_out):
        wid = lax.axis_index("s") * info.num_cores + lax.axis_index("c")
        mm = wid // wpq
        v = lax.rem(wid, wpq)
        n = nk * _NQ

        def unit(j):
            kk = j // _NQ
            h = lax.rem(j, _NQ)
            p = lax.rem(j, 2)
            t = _S * mm + lax.min(v + wpq * kk, _S - 1)
            off = pl.multiple_of(_H * h, _H)
            woff = pl.multiple_of(lax.rem(_H * h + _H, _B), 8)
            ins = [
                pltpu.make_async_copy(
                    x_hbm.at[t, pl.ds(off, _H)],
                    bufs.at[p, pl.ds(0, _H)], sem_in),
                pltpu.make_async_copy(
                    x_hbm.at[t, pl.ds(woff, 8)],
                    bufs.at[p, pl.ds(_H, 8)], sem_in),
            ]
            out = pltpu.make_async_copy(
                stage.at[p, pl.ds(0, _H)],
                out_hbm.at[t, pl.ds(off, _H)], sem_out)
            return ins, out

        def shift(p):
            def body(r, carry):
                src = bufs.at[p, r + mm]
                dst = stage.at[p, r]
                for vv in range(_NVC):
                    dst[pl.ds(vv * 16, 16)] = src[pl.ds(vv * 16, 16)]
                return carry

            lax.fori_loop(0, _H, body, 0, unroll=2)

        for c in unit(0)[0]:
            c.start()

        def pipeline_step(j, carry):
            ins_j, out_j = unit(j)
            for c in ins_j:
                c.wait()

            @pl.when(j >= 1)
            def _():
                out_j.wait()

            @pl.when(j + 1 < n)
            def _():
                for c in unit(j + 1)[0]:
                    c.start()
            shift(lax.rem(j, 2))
            out_j.start()
            return carry

        lax.fori_loop(0, n, pipeline_step, 0)
        unit(n - 1)[1].wait()

    return k(xt)


def kernel(patches, m):
    del m
    xt = jnp.transpose(patches, (1, 0, 2))
    mixed = jnp.transpose(_sc_permute_t(xt), (1, 0, 2))
    ids_b = jnp.arange(_B).reshape(-1, 1)
    target = (ids_b + jnp.arange(_M)) % _B
    mix_target = (ids_b - _M + 1 + jnp.arange(_M * 2 - 1) + _B) % _B
    return (mixed, target, mix_target)


# speedup vs baseline: 2.5387x; 2.0310x over previous
import functools

import jax
import jax.numpy as jnp
from jax import lax
from jax.experimental import pallas as pl
from jax.experimental.pallas import tpu as pltpu
from jax.experimental.pallas import tpu_sc as plsc

_B, _T, _C = 128, 196, 768
_M = 4
_S = _T // _M
_H = _B // 4
_NQ = _B // _H
_COV = _H + 8
_NVC = _C // 16


def _sc_permute_t(xt):
    info = plsc.get_sparse_core_info()
    nw = info.num_cores * info.num_subcores
    wpq = nw // _M
    nk = (_S + wpq - 1) // wpq
    mesh = plsc.VectorSubcoreMesh(core_axis_name="c", subcore_axis_name="s")

    @functools.partial(
        pl.kernel,
        out_type=jax.ShapeDtypeStruct((_T, _B, _C), jnp.float32),
        mesh=mesh,
        scratch_types=[
            pltpu.VMEM((2, _COV, _C), jnp.float32),
            pltpu.VMEM((2, _H, _C), jnp.float32),
            pltpu.SemaphoreType.DMA,
            pltpu.SemaphoreType.DMA,
        ],
    )
    def k(x_hbm, out_hbm, bufs, stage, sem_in, sem_out):
        wid = lax.axis_index("s") * info.num_cores + lax.axis_index("c")
        mm = wid // wpq
        v = lax.rem(wid, wpq)
        n = nk * _NQ

        def unit(j):
            kk = j // _NQ
            h = lax.rem(j, _NQ)
            p = lax.rem(j, 2)
            t = _S * mm + lax.min(v + wpq * kk, _S - 1)
            off = pl.multiple_of(_H * h, _H)
            woff = pl.multiple_of(lax.rem(_H * h + _H, _B), 8)
            ins = [
                pltpu.make_async_copy(
                    x_hbm.at[t, pl.ds(off, _H)],
                    bufs.at[p, pl.ds(0, _H)], sem_in),
                pltpu.make_async_copy(
                    x_hbm.at[t, pl.ds(woff, 8)],
                    bufs.at[p, pl.ds(_H, 8)], sem_in),
            ]
            out = pltpu.make_async_copy(
                stage.at[p, pl.ds(0, _H)],
                out_hbm.at[t, pl.ds(off, _H)], sem_out)
            return ins, out

        def shift(p):
            @plsc.parallel_loop(0, _H, unroll=4)
            def _(r):
                src = bufs.at[p, r + mm]
                dst = stage.at[p, r]
                for vv in range(_NVC):
                    dst[pl.ds(vv * 16, 16)] = src[pl.ds(vv * 16, 16)]

        for c in unit(0)[0]:
            c.start()

        def pipeline_step(j, carry):
            ins_j, out_j = unit(j)
            for c in ins_j:
                c.wait()

            @pl.when(j >= 1)
            def _():
                out_j.wait()

            @pl.when(j + 1 < n)
            def _():
                for c in unit(j + 1)[0]:
                    c.start()
            shift(lax.rem(j, 2))
            out_j.start()
            return carry

        lax.fori_loop(0, n, pipeline_step, 0)
        unit(n - 1)[1].wait()

    return k(xt)


def kernel(patches, m):
    del m
    xt = jnp.transpose(patches, (1, 0, 2))
    mixed = jnp.transpose(_sc_permute_t(xt), (1, 0, 2))
    ids_b = jnp.arange(_B).reshape(-1, 1)
    target = (ids_b + jnp.arange(_M)) % _B
    mix_target = (ids_b - _M + 1 + jnp.arange(_M * 2 - 1) + _B) % _B
    return (mixed, target, mix_target)
